# baseline (speedup 1.0000x reference)
import functools

import jax
import jax.numpy as jnp
from jax import lax
from jax.experimental import pallas as pl
from jax.experimental.pallas import tpu as pltpu
from jax.experimental.pallas import tpu_sc as plsc

E = 160000
EH = E // 2
C = 128
ROW_W = 4 * C
NC = 2
NS = 16
NW = NC * NS
EPW = EH // NW
BE = 25
NBLK = EPW // BE
CB = C * BE
BLKX = 4 * CB
BLKW = BE * ROW_W
BLKO = 8 * CB
EC = EH * C

_INV_SQRT3 = 0.5773502691896258


def _tp_body(x_hbm, y_hbm, w_hbm, out_hbm,
             xv, yv, wv, ov, sem_i0, sem_i1, sem_o0, sem_o1):
    wid = lax.axis_index("s") * NC + lax.axis_index("c")
    e0 = wid * EPW
    sems_i = (sem_i0, sem_i1)
    sems_o = (sem_o0, sem_o1)

    def start_in(b, j, sem):
        src = (e0 + b * BE) * C
        for d in range(4):
            pltpu.async_copy(x_hbm.at[pl.ds(d * EC + src, CB)],
                             xv.at[pl.ds(j * BLKX + d * CB, CB)], sem)
            pltpu.async_copy(y_hbm.at[pl.ds(d * EC + src, CB)],
                             yv.at[pl.ds(j * BLKX + d * CB, CB)], sem)
        pltpu.async_copy(w_hbm.at[pl.ds((e0 + b * BE) * ROW_W, BLKW)],
                         wv.at[pl.ds(j * BLKW, BLKW)], sem)

    def wait_in(j, sem):
        for _ in range(8):
            pltpu.make_async_copy(x_hbm.at[pl.ds(0, CB)],
                                  xv.at[pl.ds(j * BLKX, CB)], sem).wait()
        pltpu.make_async_copy(x_hbm.at[pl.ds(0, BLKW)],
                              wv.at[pl.ds(j * BLKW, BLKW)], sem).wait()

    def start_out(b, j, sem):
        pltpu.async_copy(ov.at[pl.ds(j * BLKO, BLKO)],
                         out_hbm.at[pl.ds((e0 + b * BE) * 8 * C, BLKO)], sem)

    def wait_out(j, sem):
        pltpu.make_async_copy(ov.at[pl.ds(j * BLKO, BLKO)],
                              out_hbm.at[pl.ds(0, BLKO)], sem).wait()

    def compute_block(j):
        @plsc.parallel_loop(0, BE, 1)
        def edge(e):
            bx = j * BLKX + e * C
            bw = j * BLKW + e * ROW_W
            bo = j * BLKO + e * 8 * C
            for g in range(8):
                c0 = 16 * g
                X = [xv[pl.ds(bx + d * CB + c0, 16)] for d in range(4)]
                Y = [yv[pl.ds(bx + d * CB + c0, 16)] for d in range(4)]
                W = [wv[pl.ds(bw + s * C + c0, 16)] for s in range(4)]
                ov[pl.ds(bo + 0 * C + c0, 16)] = W[0] * X[0] * Y[0]
                ov[pl.ds(bo + 1 * C + c0, 16)] = (W[1] * _INV_SQRT3) * (
                    X[1] * Y[1] + X[2] * Y[2] + X[3] * Y[3])
                xw2 = W[2] * X[0]
                yw3 = W[3] * Y[0]
                for jj in range(3):
                    ov[pl.ds(bo + (2 + jj) * C + c0, 16)] = xw2 * Y[1 + jj]
                    ov[pl.ds(bo + (5 + jj) * C + c0, 16)] = yw3 * X[1 + jj]

    start_in(0, 0, sems_i[0])
    start_in(1, 1, sems_i[1])

    def superblock(sb, carry):
        b0 = 2 * sb
        for j in range(2):
            b = b0 + j
            wait_in(j, sems_i[j])

            @pl.when(b >= 2)
            def _():
                wait_out(j, sems_o[j])

            compute_block(j)
            start_out(b, j, sems_o[j])

            @pl.when(b + 2 < NBLK)
            def _():
                start_in(b + 2, j, sems_i[j])
        return carry

    lax.fori_loop(0, NBLK // 2, superblock, 0)
    wait_out(0, sems_o[0])
    wait_out(1, sems_o[1])


@functools.partial(jax.jit)
def _tp_sc(xt, yt, wf):
    mesh = plsc.VectorSubcoreMesh(core_axis_name="c", subcore_axis_name="s")
    f = functools.partial(
        pl.kernel,
        out_type=jax.ShapeDtypeStruct((8 * EC,), jnp.float32),
        mesh=mesh,
        compiler_params=pltpu.CompilerParams(
            needs_layout_passes=False, use_tc_tiling_on_sc=True),
        scratch_types=[
            pltpu.VMEM((2 * BLKX,), jnp.float32),
            pltpu.VMEM((2 * BLKX,), jnp.float32),
            pltpu.VMEM((2 * BLKW,), jnp.float32),
            pltpu.VMEM((2 * BLKO,), jnp.float32),
            pltpu.SemaphoreType.DMA,
            pltpu.SemaphoreType.DMA,
            pltpu.SemaphoreType.DMA,
            pltpu.SemaphoreType.DMA,
        ],
    )(_tp_body)
    return f(xt, yt, wf)


def kernel(x, y, weights):
    eye_b, eye_k = lax.optimization_barrier(
        (jnp.broadcast_to(jnp.eye(C, dtype=jnp.float32), (4, C, C)),
         jnp.eye(8, dtype=jnp.float32)))
    outs = []
    for h in range(2):
        xs = lax.slice_in_dim(x, h * EH, (h + 1) * EH, axis=0)
        ys = lax.slice_in_dim(y, h * EH, (h + 1) * EH, axis=0)
        ws = lax.slice_in_dim(weights, h * EH, (h + 1) * EH, axis=0)
        xt = jnp.einsum('ecd,dcf->def', xs, eye_b,
                        precision=lax.Precision.HIGH)
        yt = jnp.einsum('ecd,dcf->def', ys, eye_b,
                        precision=lax.Precision.HIGH)
        out8 = _tp_sc(xt.reshape(-1), yt.reshape(-1), ws.reshape(-1))
        outs.append(jnp.einsum('ekc,kj->ecj', out8.reshape(EH, 8, C), eye_k,
                               precision=lax.Precision.HIGH))
    return jnp.concatenate(outs, axis=0)


# speedup vs baseline: 1.5465x; 1.5465x over previous
import functools

import jax
import jax.numpy as jnp
from jax import lax
from jax.experimental import pallas as pl
from jax.experimental.pallas import tpu as pltpu
from jax.experimental.pallas import tpu_sc as plsc

E = 160000
C = 128
ROW_W = 4 * C
NC = 2
NS = 16
NW = NC * NS
EPW = E // NW
BE = 20
NBLK = EPW // BE
CB = C * BE
BLKX = 4 * CB
BLKW = BE * ROW_W
BLKO = 8 * CB
EC = E * C

_INV_SQRT3 = 0.5773502691896258


def _tp_body(x_hbm, y_hbm, w_hbm, out_hbm,
             xv, yv, wv, ov, sem_i0, sem_i1, sem_o0, sem_o1):
    wid = lax.axis_index("s") * NC + lax.axis_index("c")
    e0 = wid * EPW
    sems_i = (sem_i0, sem_i1)
    sems_o = (sem_o0, sem_o1)

    def start_in(b, j, sem):
        src = (e0 + b * BE) * C
        for d in range(4):
            pltpu.async_copy(x_hbm.at[pl.ds(d * EC + src, CB)],
                             xv.at[pl.ds(j * BLKX + d * CB, CB)], sem)
            pltpu.async_copy(y_hbm.at[pl.ds(d * EC + src, CB)],
                             yv.at[pl.ds(j * BLKX + d * CB, CB)], sem)
        pltpu.async_copy(w_hbm.at[pl.ds((e0 + b * BE) * ROW_W, BLKW)],
                         wv.at[pl.ds(j * BLKW, BLKW)], sem)

    def wait_in(j, sem):
        for _ in range(8):
            pltpu.make_async_copy(x_hbm.at[pl.ds(0, CB)],
                                  xv.at[pl.ds(j * BLKX, CB)], sem).wait()
        pltpu.make_async_copy(x_hbm.at[pl.ds(0, BLKW)],
                              wv.at[pl.ds(j * BLKW, BLKW)], sem).wait()

    def start_out(b, j, sem):
        pltpu.async_copy(ov.at[pl.ds(j * BLKO, BLKO)],
                         out_hbm.at[pl.ds((e0 + b * BE) * 8 * C, BLKO)], sem)

    def wait_out(j, sem):
        pltpu.make_async_copy(ov.at[pl.ds(j * BLKO, BLKO)],
                              out_hbm.at[pl.ds(0, BLKO)], sem).wait()

    def compute_block(j):
        @plsc.parallel_loop(0, BE, 1)
        def edge(e):
            bx = j * BLKX + e * C
            bw = j * BLKW + e * ROW_W
            bo = j * BLKO + e * 8 * C
            for g in range(8):
                c0 = 16 * g
                X = [xv[pl.ds(bx + d * CB + c0, 16)] for d in range(4)]
                Y = [yv[pl.ds(bx + d * CB + c0, 16)] for d in range(4)]
                W = [wv[pl.ds(bw + s * C + c0, 16)] for s in range(4)]
                ov[pl.ds(bo + 0 * C + c0, 16)] = W[0] * X[0] * Y[0]
                ov[pl.ds(bo + 1 * C + c0, 16)] = (W[1] * _INV_SQRT3) * (
                    X[1] * Y[1] + X[2] * Y[2] + X[3] * Y[3])
                xw2 = W[2] * X[0]
                yw3 = W[3] * Y[0]
                for jj in range(3):
                    ov[pl.ds(bo + (2 + jj) * C + c0, 16)] = xw2 * Y[1 + jj]
                    ov[pl.ds(bo + (5 + jj) * C + c0, 16)] = yw3 * X[1 + jj]

    start_in(0, 0, sems_i[0])
    start_in(1, 1, sems_i[1])

    def superblock(sb, carry):
        b0 = 2 * sb
        for j in range(2):
            b = b0 + j
            wait_in(j, sems_i[j])

            @pl.when(b >= 2)
            def _():
                wait_out(j, sems_o[j])

            compute_block(j)
            start_out(b, j, sems_o[j])

            @pl.when(b + 2 < NBLK)
            def _():
                start_in(b + 2, j, sems_i[j])
        return carry

    lax.fori_loop(0, NBLK // 2, superblock, 0)
    wait_out(0, sems_o[0])
    wait_out(1, sems_o[1])


@functools.partial(jax.jit)
def _tp_sc(xt, yt, wf):
    mesh = plsc.VectorSubcoreMesh(core_axis_name="c", subcore_axis_name="s")
    f = functools.partial(
        pl.kernel,
        out_type=jax.ShapeDtypeStruct((8 * EC,), jnp.float32),
        mesh=mesh,
        compiler_params=pltpu.CompilerParams(
            needs_layout_passes=False, use_tc_tiling_on_sc=True),
        scratch_types=[
            pltpu.VMEM((2 * BLKX,), jnp.float32),
            pltpu.VMEM((2 * BLKX,), jnp.float32),
            pltpu.VMEM((2 * BLKW,), jnp.float32),
            pltpu.VMEM((2 * BLKO,), jnp.float32),
            pltpu.SemaphoreType.DMA,
            pltpu.SemaphoreType.DMA,
            pltpu.SemaphoreType.DMA,
            pltpu.SemaphoreType.DMA,
        ],
    )(_tp_body)
    return f(xt, yt, wf)


def kernel(x, y, weights):
    eye_b, eye_k = lax.optimization_barrier(
        (jnp.broadcast_to(jnp.eye(C, dtype=jnp.float32), (4, C, C)),
         jnp.eye(8, dtype=jnp.float32)))
    xt = jnp.einsum('ecd,dcf->def', x, eye_b,
                    precision=lax.Precision.HIGH)
    yt = jnp.einsum('ecd,dcf->def', y, eye_b,
                    precision=lax.Precision.HIGH)
    wf = weights.reshape(-1)
    out8 = _tp_sc(xt.reshape(-1), yt.reshape(-1), wf)
    out8 = out8.reshape(E, 8, C)
    return jnp.einsum('ekc,kj->ecj', out8, eye_k,
                      precision=lax.Precision.HIGH)
